# Initial kernel scaffold; baseline (speedup 1.0000x reference)
import jax
import jax.numpy as jnp
from jax.experimental import pallas as pl


def kernel(points, knn_graph, kernel):
    raise NotImplementedError("write your pallas kernel here")


# speedup vs baseline: 55.1864x; 55.1864x over previous
import functools

import jax
import jax.numpy as jnp
from jax import lax
from jax.experimental import pallas as pl
from jax.experimental.pallas import tpu as pltpu
from jax.experimental.pallas import tpu_sc as plsc

B = 8
C = 3
N = 16384
K = 8
L = 8
M = 8

NW = 32
WPB = NW // B
NPW = N // WPB
SUB = 512
NB = 512


def _sc_gather_body(points_hbm, knn_hbm, xc_hbm, px_v, py_v, pz_v, knn_v, out_v):
    wid = lax.axis_index("s") * 2 + lax.axis_index("c")
    b = wid // WPB
    n0 = (wid % WPB) * NPW

    pltpu.sync_copy(points_hbm.at[pl.ds((b * C + 0) * N, N)], px_v)
    pltpu.sync_copy(points_hbm.at[pl.ds((b * C + 1) * N, N)], py_v)
    pltpu.sync_copy(points_hbm.at[pl.ds((b * C + 2) * N, N)], pz_v)
    pltpu.sync_copy(knn_hbm.at[pl.ds((b * N + n0) * K, NPW * K)], knn_v)

    lane8 = lax.iota(jnp.int32, 16) * K

    def group(g, s_base):
        local = s_base + g * 16
        gbase = n0 + local
        cx = px_v[pl.ds(gbase, 16)]
        cy = py_v[pl.ds(gbase, 16)]
        cz = pz_v[pl.ds(gbase, 16)]
        off = g * 16
        for k in range(K):
            idx = plsc.load_gather(knn_v, [lane8 + (local * K + k)])
            out_v[0, k, pl.ds(off, 16)] = plsc.load_gather(px_v, [idx]) - cx
            out_v[1, k, pl.ds(off, 16)] = plsc.load_gather(py_v, [idx]) - cy
            out_v[2, k, pl.ds(off, 16)] = plsc.load_gather(pz_v, [idx]) - cz

    for s in range(NPW // SUB):
        s_base = s * SUB
        lax.fori_loop(0, SUB // 16, lambda g, _: (group(g, s_base), 0)[1], 0)
        pltpu.sync_copy(out_v, xc_hbm.at[b, :, :, pl.ds(n0 + s_base, SUB)])


_sc_gather = functools.partial(
    pl.kernel,
    out_type=jax.ShapeDtypeStruct((B, C, K, N), jnp.float32),
    mesh=plsc.VectorSubcoreMesh(core_axis_name="c", subcore_axis_name="s"),
    scratch_types=[
        pltpu.VMEM((N,), jnp.float32),
        pltpu.VMEM((N,), jnp.float32),
        pltpu.VMEM((N,), jnp.float32),
        pltpu.VMEM((NPW * K,), jnp.int32),
        pltpu.VMEM((C, K, SUB), jnp.float32),
    ],
    compiler_params=pltpu.CompilerParams(needs_layout_passes=False),
)(_sc_gather_body)


def _tc_body(kp_ref, xc_ref, out_ref):
    x0 = xc_ref[0, 0]
    x1 = xc_ref[0, 1]
    x2 = xc_ref[0, 2]
    a = -2.0 * (x0 * x0 + x1 * x1 + x2 * x2)
    rows = []
    for l in range(L):
        acc = None
        for m in range(M):
            j = l * M + m
            e = jnp.exp(a + kp_ref[0, j] * x0 + kp_ref[1, j] * x1
                        + kp_ref[2, j] * x2 + kp_ref[3, j])
            acc = e if acc is None else acc + e
        rows.append(jnp.sum(acc, axis=0, keepdims=True) * (1.0 / K))
    out_ref[0] = jnp.concatenate(rows, axis=0)


def _tc_compute(kp, xc):
    return pl.pallas_call(
        _tc_body,
        grid=(B, N // NB),
        in_specs=[
            pl.BlockSpec((4, L * M), lambda b, n: (0, 0), memory_space=pltpu.SMEM),
            pl.BlockSpec((1, C, K, NB), lambda b, n: (b, 0, 0, n)),
        ],
        out_specs=pl.BlockSpec((1, L, NB), lambda b, n: (b, 0, n)),
        out_shape=jax.ShapeDtypeStruct((B, L, N), jnp.float32),
    )(kp, xc)


def kernel(points, knn_graph, kernel):
    xc = _sc_gather(points.reshape(-1), knn_graph.reshape(-1))
    kf = kernel.reshape(L * M, C)
    kp = jnp.stack([4.0 * kf[:, 0], 4.0 * kf[:, 1], 4.0 * kf[:, 2],
                    -2.0 * jnp.sum(kf * kf, axis=1)], axis=0)
    return _tc_compute(kp, xc)
